# Initial kernel scaffold; baseline (speedup 1.0000x reference)
import jax
import jax.numpy as jnp
from jax.experimental import pallas as pl


def kernel(x, edge_index, W1, att_src1, att_dst1, b1, W2, att_src2, att_dst2, b2, W3, att_src3, att_dst3, b3):
    raise NotImplementedError("write your pallas kernel here")


# speedup vs baseline: 1.0960x; 1.0960x over previous
import jax
import jax.numpy as jnp
from jax.experimental import pallas as pl

N_NODES = 10000
HEADS = 8
HEAD_DIM = 64


def _matmul_kernel(x_ref, w_ref, o_ref):
    o_ref[...] = jnp.dot(x_ref[...], w_ref[...],
                         preferred_element_type=jnp.float32)


def _matmul(x, w, bn=400):
    n, k = x.shape
    m = w.shape[1]
    grid = (n // bn,)
    return pl.pallas_call(
        _matmul_kernel,
        grid=grid,
        in_specs=[pl.BlockSpec((bn, k), lambda i: (i, 0)),
                  pl.BlockSpec((k, m), lambda i: (0, 0))],
        out_specs=pl.BlockSpec((bn, m), lambda i: (i, 0)),
        out_shape=jax.ShapeDtypeStruct((n, m), jnp.float32),
    )(x, w)


def _gat_conv(x, edge_index, W, att_src, att_dst, bias, heads, out_ch, concat):
    N = x.shape[0]
    loop = jnp.arange(N, dtype=edge_index.dtype)
    src = jnp.concatenate([edge_index[0], loop])
    dst = jnp.concatenate([edge_index[1], loop])
    h = _matmul(x, W).reshape(N, heads, out_ch)
    a_src = (h * att_src[None, :, :]).sum(-1)
    a_dst = (h * att_dst[None, :, :]).sum(-1)
    alpha = a_src[src] + a_dst[dst]
    alpha = jax.nn.leaky_relu(alpha, negative_slope=0.2)
    alpha = jnp.exp(alpha)
    denom = jax.ops.segment_sum(alpha, dst, num_segments=N)
    msg = h[src] * alpha[:, :, None]
    out = jax.ops.segment_sum(msg, dst, num_segments=N)
    out = out / (denom[:, :, None] + 1e-16)
    if concat:
        out = out.reshape(N, heads * out_ch)
    else:
        out = out.mean(axis=1)
    return out + bias


def kernel(x, edge_index, W1, att_src1, att_dst1, b1, W2, att_src2, att_dst2, b2,
           W3, att_src3, att_dst3, b3):
    h = _gat_conv(x, edge_index, W1, att_src1, att_dst1, b1, HEADS, HEAD_DIM, True)
    h = jax.nn.elu(h)
    h = _gat_conv(h, edge_index, W2, att_src2, att_dst2, b2, HEADS, HEAD_DIM, True)
    h = jax.nn.elu(h)
    out = _gat_conv(h, edge_index, W3, att_src3, att_dst3, b3, 1, 64, False)
    return out


# speedup vs baseline: 17.5105x; 15.9773x over previous
import functools

import jax
import jax.numpy as jnp
from jax import lax
from jax.experimental import pallas as pl
from jax.experimental.pallas import tpu as pltpu
from jax.experimental.pallas import tpu_sc as plsc

N = 10000
N_PAD = 10240
TRASH = N
E_TOT = 160000 + N
E_PAD = 172032
CHUNK = 128
ROWS_PER_TEC = N_PAD // 16
F32 = jnp.float32

_mesh = lambda: plsc.VectorSubcoreMesh(core_axis_name="c", subcore_axis_name="s")


def _split_cols(hb):
    return hb[:, 0:128], hb[:, 128:256], hb[:, 256:384], hb[:, 384:512]


def _tc1_body(x_ref, w_ref, p_ref, h0, h1, h2, h3, ast, adt):
    hb = jnp.dot(x_ref[...], w_ref[...], preferred_element_type=F32)
    ab = jnp.dot(hb, p_ref[...], preferred_element_type=F32)
    h0[...], h1[...], h2[...], h3[...] = _split_cols(hb)
    z = jnp.zeros((hb.shape[0], 8), F32)
    ast[...] = jnp.concatenate([ab[:, 0:8], z], axis=1)
    adt[...] = jnp.concatenate([ab[:, 8:16], z], axis=1)


def _tc1(x_pad, W1, P1, bn=256):
    grid = (N_PAD // bn,)
    fs = jax.ShapeDtypeStruct
    return pl.pallas_call(
        _tc1_body,
        grid=grid,
        in_specs=[pl.BlockSpec((bn, 256), lambda i: (i, 0)),
                  pl.BlockSpec((256, 512), lambda i: (0, 0)),
                  pl.BlockSpec((512, 16), lambda i: (0, 0))],
        out_specs=[pl.BlockSpec((bn, 128), lambda i: (i, 0))] * 4
                  + [pl.BlockSpec((bn, 16), lambda i: (i, 0))] * 2,
        out_shape=[fs((N_PAD, 128), F32)] * 4 + [fs((N_PAD, 16), F32)] * 2,
    )(x_pad, W1, P1)


def _tc_mid_body(o0, o1, o2, o3, den, b_ref, w_ref, p_ref, *outs):
    bn = o0.shape[0]
    ob = jnp.concatenate([o0[...], o1[...], o2[...], o3[...]], axis=1)
    d8 = den[...][:, 0:8]
    div = jnp.broadcast_to(d8[:, :, None], (bn, 8, 64)).reshape(bn, 512) + 1e-16
    sb = ob / div + b_ref[...]
    sb = jnp.where(sb > 0, sb, jnp.exp(sb) - 1.0)
    hb = jnp.dot(sb, w_ref[...], preferred_element_type=F32)
    ab = jnp.dot(hb, p_ref[...], preferred_element_type=F32)
    if len(outs) == 6:
        h0, h1, h2, h3, ast, adt = outs
        h0[...], h1[...], h2[...], h3[...] = _split_cols(hb)
        z = jnp.zeros((bn, 8), F32)
        ast[...] = jnp.concatenate([ab[:, 0:8], z], axis=1)
        adt[...] = jnp.concatenate([ab[:, 8:16], z], axis=1)
    else:
        h3_, ast, adt = outs
        h3_[...] = hb
        z15 = jnp.zeros((bn, 15), F32)
        ast[...] = jnp.concatenate([ab[:, 0:1], z15], axis=1)
        adt[...] = jnp.concatenate([ab[:, 8:9], z15], axis=1)


def _tc_mid(og, den, b, W, P, n_out, bn=256):
    grid = (N_PAD // bn,)
    fs = jax.ShapeDtypeStruct
    m = W.shape[1]
    if n_out == 512:
        out_specs = ([pl.BlockSpec((bn, 128), lambda i: (i, 0))] * 4
                     + [pl.BlockSpec((bn, 16), lambda i: (i, 0))] * 2)
        out_shape = [fs((N_PAD, 128), F32)] * 4 + [fs((N_PAD, 16), F32)] * 2
    else:
        out_specs = ([pl.BlockSpec((bn, 64), lambda i: (i, 0))]
                     + [pl.BlockSpec((bn, 16), lambda i: (i, 0))] * 2)
        out_shape = [fs((N_PAD, 64), F32)] + [fs((N_PAD, 16), F32)] * 2
    return pl.pallas_call(
        _tc_mid_body,
        grid=grid,
        in_specs=[pl.BlockSpec((bn, 128), lambda i: (i, 0))] * 4
                 + [pl.BlockSpec((bn, 16), lambda i: (i, 0)),
                    pl.BlockSpec((1, 512), lambda i: (0, 0)),
                    pl.BlockSpec((512, m), lambda i: (0, 0)),
                    pl.BlockSpec((m, 16), lambda i: (0, 0))],
        out_specs=out_specs,
        out_shape=out_shape,
    )(*og, den, b, W, P)


def _tc_final_body(p0, p1, d0, d1, b_ref, out_ref):
    den = d0[...][:, 0:1] + d1[...][:, 0:1] + 1e-16
    out_ref[...] = (p0[...] + p1[...]) / den + b_ref[...]


def _tc_final(p0, p1, d0, d1, b3, bn=400):
    grid = (N // bn,)
    return pl.pallas_call(
        _tc_final_body,
        grid=grid,
        in_specs=[pl.BlockSpec((bn, 64), lambda i: (i, 0))] * 2
                 + [pl.BlockSpec((bn, 16), lambda i: (i, 0))] * 2
                 + [pl.BlockSpec((1, 64), lambda i: (0, 0))],
        out_specs=pl.BlockSpec((bn, 64), lambda i: (i, 0)),
        out_shape=jax.ShapeDtypeStruct((N, 64), F32),
    )(p0, p1, d0, d1, b3)


_ZERO16 = None


def _zero_rows(buf, width):
    z = jnp.zeros((16,), F32)

    def body(r, _):
        for j in range(width // 16):
            buf[r, pl.ds(j * 16, 16)] = z
        return 0

    lax.fori_loop(0, CHUNK, body, 0, unroll=4)


def _zero_shared(acc, buf, s, width):
    for k in range(ROWS_PER_TEC // CHUNK):
        pltpu.sync_copy(buf, acc.at[pl.ds(s * ROWS_PER_TEC + k * CHUNK, CHUNK)])


def _sc12_build():
    fs = jax.ShapeDtypeStruct
    per_tec = E_PAD // 16

    @functools.partial(
        pl.kernel,
        out_type=[fs((N_PAD, 128), F32)] * 4 + [fs((N_PAD, 16), F32)],
        mesh=_mesh(),
        compiler_params=pltpu.CompilerParams(use_tc_tiling_on_sc=False),
        scratch_types=[
            pltpu.VMEM((CHUNK,), jnp.int32),
            pltpu.VMEM((CHUNK,), jnp.int32),
            pltpu.VMEM((CHUNK, 16), F32),
            pltpu.VMEM((CHUNK, 16), F32),
            pltpu.VMEM((CHUNK, 16), F32),
            pltpu.VMEM((CHUNK, 128), F32),
            pltpu.VMEM_SHARED((N_PAD, 128), F32),
            pltpu.VMEM_SHARED((N_PAD, 16), F32),
            pltpu.SemaphoreType.DMA,
            pltpu.SemaphoreType.DMA,
            pltpu.SemaphoreType.DMA,
        ],
    )
    def sc12(src_hbm, dst_hbm, ast_hbm, adt_hbm, hg0, hg1, hg2, hg3,
             og0, og1, og2, og3, den_out,
             sidx, didx, sbuf, dbuf, abuf, hbuf, acc, dacc, s0, s1, s2):
        c = lax.axis_index("c")
        s = lax.axis_index("s")

        _zero_rows(hbuf, 128)
        _zero_shared(acc, hbuf, s, 128)
        _zero_rows(sbuf, 16)
        _zero_shared(dacc, sbuf, s, 16)
        plsc.subcore_barrier()

        def edge_pass(hg, g, add_denom):
            a_col = 2 * g

            def chunk_body(t, _):
                base = s * per_tec + t * CHUNK
                pltpu.sync_copy(src_hbm.at[pl.ds(base, CHUNK)], sidx)
                pltpu.sync_copy(dst_hbm.at[pl.ds(base, CHUNK)], didx)
                cp_s = pltpu.async_copy(ast_hbm.at[sidx], sbuf, s0)
                cp_d = pltpu.async_copy(adt_hbm.at[didx], dbuf, s1)
                cp_h = pltpu.async_copy(hg.at[sidx], hbuf, s2)
                cp_s.wait()
                cp_d.wait()

                def alpha_body(e, _):
                    av = sbuf[e, :] + dbuf[e, :]
                    av = jnp.where(av > 0, av, 0.2 * av)
                    abuf[e, :] = jnp.exp(av)
                    return 0

                lax.fori_loop(0, CHUNK, alpha_body, 0, unroll=4)
                if add_denom:
                    pltpu.sync_copy(abuf, dacc.at[didx], add=True)
                cp_h.wait()

                def scale_body(e, _):
                    av = abuf[e, :]
                    a0 = av[a_col]
                    a1 = av[a_col + 1]
                    for j in range(8):
                        aa = a0 if j < 4 else a1
                        hbuf[e, pl.ds(j * 16, 16)] = hbuf[e, pl.ds(j * 16, 16)] * aa
                    return 0

                lax.fori_loop(0, CHUNK, scale_body, 0, unroll=2)
                pltpu.sync_copy(hbuf, acc.at[didx], add=True)
                return 0

            lax.fori_loop(0, per_tec // CHUNK, chunk_body, 0)

        def dump(og):
            pltpu.sync_copy(acc.at[pl.ds(s * ROWS_PER_TEC, ROWS_PER_TEC)],
                            og.at[pl.ds(s * ROWS_PER_TEC, ROWS_PER_TEC)])

        def rezero_acc():
            _zero_rows(hbuf, 128)
            _zero_shared(acc, hbuf, s, 128)

        @pl.when(c == 0)
        def _():
            edge_pass(hg0, 0, True)
            plsc.subcore_barrier()
            dump(og0)
            pltpu.sync_copy(dacc.at[pl.ds(s * ROWS_PER_TEC, ROWS_PER_TEC)],
                            den_out.at[pl.ds(s * ROWS_PER_TEC, ROWS_PER_TEC)])
            rezero_acc()
            plsc.subcore_barrier()
            edge_pass(hg1, 1, False)
            plsc.subcore_barrier()
            dump(og1)

        @pl.when(c == 1)
        def _():
            edge_pass(hg2, 2, False)
            plsc.subcore_barrier()
            dump(og2)
            rezero_acc()
            plsc.subcore_barrier()
            edge_pass(hg3, 3, False)
            plsc.subcore_barrier()
            dump(og3)

    return sc12


def _sc3_build():
    fs = jax.ShapeDtypeStruct
    per_tec = E_PAD // 32

    @functools.partial(
        pl.kernel,
        out_type=[fs((2, N_PAD, 64), F32), fs((2, N_PAD, 16), F32)],
        mesh=_mesh(),
        compiler_params=pltpu.CompilerParams(use_tc_tiling_on_sc=False),
        scratch_types=[
            pltpu.VMEM((CHUNK,), jnp.int32),
            pltpu.VMEM((CHUNK,), jnp.int32),
            pltpu.VMEM((CHUNK, 16), F32),
            pltpu.VMEM((CHUNK, 16), F32),
            pltpu.VMEM((CHUNK, 16), F32),
            pltpu.VMEM((CHUNK, 64), F32),
            pltpu.VMEM_SHARED((N_PAD, 64), F32),
            pltpu.VMEM_SHARED((N_PAD, 16), F32),
            pltpu.SemaphoreType.DMA,
            pltpu.SemaphoreType.DMA,
            pltpu.SemaphoreType.DMA,
        ],
    )
    def sc3(src_hbm, dst_hbm, ast_hbm, adt_hbm, h3_hbm,
            op, dp,
            sidx, didx, sbuf, dbuf, abuf, hbuf, acc, dacc, s0, s1, s2):
        c = lax.axis_index("c")
        s = lax.axis_index("s")

        _zero_rows(hbuf, 64)
        _zero_shared(acc, hbuf, s, 64)
        _zero_rows(sbuf, 16)
        _zero_shared(dacc, sbuf, s, 16)
        plsc.subcore_barrier()

        def chunk_body(t, _):
            base = (c * 16 + s) * per_tec + t * CHUNK
            pltpu.sync_copy(src_hbm.at[pl.ds(base, CHUNK)], sidx)
            pltpu.sync_copy(dst_hbm.at[pl.ds(base, CHUNK)], didx)
            cp_s = pltpu.async_copy(ast_hbm.at[sidx], sbuf, s0)
            cp_d = pltpu.async_copy(adt_hbm.at[didx], dbuf, s1)
            cp_h = pltpu.async_copy(h3_hbm.at[sidx], hbuf, s2)
            cp_s.wait()
            cp_d.wait()

            def alpha_body(e, _):
                av = sbuf[e, :] + dbuf[e, :]
                av = jnp.where(av > 0, av, 0.2 * av)
                abuf[e, :] = jnp.exp(av)
                return 0

            lax.fori_loop(0, CHUNK, alpha_body, 0, unroll=4)
            pltpu.sync_copy(abuf, dacc.at[didx], add=True)
            cp_h.wait()

            def scale_body(e, _):
                a0 = abuf[e, :][0]
                for j in range(4):
                    hbuf[e, pl.ds(j * 16, 16)] = hbuf[e, pl.ds(j * 16, 16)] * a0
                return 0

            lax.fori_loop(0, CHUNK, scale_body, 0, unroll=2)
            pltpu.sync_copy(hbuf, acc.at[didx], add=True)
            return 0

        lax.fori_loop(0, per_tec // CHUNK, chunk_body, 0)
        plsc.subcore_barrier()
        pltpu.sync_copy(acc.at[pl.ds(s * ROWS_PER_TEC, ROWS_PER_TEC)],
                        op.at[c, pl.ds(s * ROWS_PER_TEC, ROWS_PER_TEC)])
        pltpu.sync_copy(dacc.at[pl.ds(s * ROWS_PER_TEC, ROWS_PER_TEC)],
                        dp.at[c, pl.ds(s * ROWS_PER_TEC, ROWS_PER_TEC)])

    return sc3


def _bp(att):
    H, C = att.shape
    eye = jnp.eye(H, dtype=att.dtype)
    return (att[:, :, None] * eye[:, None, :]).reshape(H * C, H)


def _build_p(att_s, att_d):
    H = att_s.shape[0]
    ps, pd = _bp(att_s), _bp(att_d)
    z = jnp.zeros((ps.shape[0], 8 - H), att_s.dtype)
    return jnp.concatenate([ps, z, pd, z], axis=1)


def kernel(x, edge_index, W1, att_src1, att_dst1, b1,
           W2, att_src2, att_dst2, b2,
           W3, att_src3, att_dst3, b3):
    ei = edge_index.astype(jnp.int32)
    loop = jnp.arange(N, dtype=jnp.int32)
    pad_n = E_PAD - E_TOT
    src = jnp.concatenate([ei[0], loop, jnp.zeros((pad_n,), jnp.int32)])
    dst = jnp.concatenate([ei[1], loop, jnp.full((pad_n,), TRASH, jnp.int32)])
    x_pad = jnp.pad(x, ((0, N_PAD - N), (0, 0)))

    P1 = _build_p(att_src1, att_dst1)
    P2 = _build_p(att_src2, att_dst2)
    P3 = _build_p(att_src3, att_dst3)

    sc12 = _sc12_build()
    sc3 = _sc3_build()

    h0, h1, h2, h3_, ast, adt = _tc1(x_pad, W1, P1)
    og0, og1, og2, og3, den = sc12(src, dst, ast, adt, h0, h1, h2, h3_)
    h0, h1, h2, h3_, ast, adt = _tc_mid((og0, og1, og2, og3), den,
                                        b1.reshape(1, 512), W2, P2, 512)
    og0, og1, og2, og3, den = sc12(src, dst, ast, adt, h0, h1, h2, h3_)
    ht, ast, adt = _tc_mid((og0, og1, og2, og3), den,
                           b2.reshape(1, 512), W3, P3, 64)
    op, dp = sc3(src, dst, ast, adt, ht)
    return _tc_final(op[0], op[1], dp[0], dp[1], b3.reshape(1, 64))
